# Initial kernel scaffold; baseline (speedup 1.0000x reference)
import jax
import jax.numpy as jnp
from jax.experimental import pallas as pl


def kernel(z, W_dec, codebook):
    raise NotImplementedError("write your pallas kernel here")


# speedup vs baseline: 22.1215x; 22.1215x over previous
import functools

import jax
import jax.numpy as jnp
from jax.experimental import pallas as pl

_BM = 2048


def _vq_argmin_kernel(z_ref, cb_ref, out_ref):
    z = z_ref[...]
    cb = cb_ref[...]
    zsq = jnp.sum(z * z, axis=1, keepdims=True)
    csq = jnp.sum(cb * cb, axis=1)
    dots = jax.lax.dot_general(
        z, cb, (((1,), (1,)), ((), ())),
        preferred_element_type=jnp.float32)
    dist = zsq + csq[None, :] - 2.0 * dots
    out_ref[...] = jnp.argmin(dist, axis=1).astype(jnp.int32)


@functools.partial(jax.jit, static_argnames=())
def kernel(z, W_dec, codebook):
    del W_dec
    n, d = z.shape
    k = codebook.shape[0]
    grid = n // _BM
    return pl.pallas_call(
        _vq_argmin_kernel,
        grid=(grid,),
        in_specs=[
            pl.BlockSpec((_BM, d), lambda i: (i, 0)),
            pl.BlockSpec((k, d), lambda i: (0, 0)),
        ],
        out_specs=pl.BlockSpec((_BM,), lambda i: (i,)),
        out_shape=jax.ShapeDtypeStruct((n,), jnp.int32),
    )(z, codebook)


# speedup vs baseline: 25.4635x; 1.1511x over previous
import functools

import jax
import jax.numpy as jnp
from jax.experimental import pallas as pl

_BM = 1024


def _vq_argmin_kernel(z_ref, cb_ref, out_ref):
    z = z_ref[...]
    cb = cb_ref[...]
    zsq = jnp.sum(z * z, axis=1, keepdims=True)
    csq = jnp.sum(cb * cb, axis=1)
    dots = jax.lax.dot_general(
        z, cb, (((1,), (1,)), ((), ())),
        preferred_element_type=jnp.float32)
    dist = zsq + csq[None, :] - 2.0 * dots
    out_ref[...] = jnp.argmin(dist, axis=1).astype(jnp.int32)


@functools.partial(jax.jit, static_argnames=())
def kernel(z, W_dec, codebook):
    del W_dec
    n, d = z.shape
    k = codebook.shape[0]
    grid = n // _BM
    return pl.pallas_call(
        _vq_argmin_kernel,
        grid=(grid,),
        in_specs=[
            pl.BlockSpec((_BM, d), lambda i: (i, 0)),
            pl.BlockSpec((k, d), lambda i: (0, 0)),
        ],
        out_specs=pl.BlockSpec((_BM,), lambda i: (i,)),
        out_shape=jax.ShapeDtypeStruct((n,), jnp.int32),
    )(z, codebook)
